# baseline (speedup 1.0000x reference)
import functools

import jax
import jax.numpy as jnp
from jax import lax
from jax.experimental import pallas as pl
from jax.experimental.pallas import tpu as pltpu

B = 2560
C = 128
NSEG = 10000


def _pad_rows(nseg):
    need = nseg + C + 8
    return ((need + B - 1) // B) * B


def _k1_body(nblk, pad, bases_ref, nch_ref, x_ref, brow_ref, wlt_ref,
             out_ref, xb_ref, acc_ref):
    k = pl.program_id(0)

    @pl.when(k == 0)
    def _():
        acc_ref[...] = jnp.zeros_like(acc_ref)

    base = bases_ref[k]
    nch = nch_ref[k]
    brow = brow_ref[0]
    x_blk = x_ref[...].astype(jnp.bfloat16)
    xb_ref[...] = x_blk

    def chunk(j, _):
        cb = base + j * C
        rows = lax.broadcasted_iota(jnp.int32, (C, B), 0) + cb
        ohT = jnp.where(rows == brow, 1.0, 0.0).astype(jnp.bfloat16)
        acc_ref[pl.ds(cb, C), :] += jnp.dot(
            ohT, x_blk, preferred_element_type=jnp.float32)
        return 0

    lax.fori_loop(0, nch, chunk, 0)

    @pl.when(k == nblk - 1)
    def _():
        def mm(i, _):
            out_ref[pl.ds(i * B, B), :] = jnp.dot(
                acc_ref[pl.ds(i * B, B), :], wlt_ref[...],
                preferred_element_type=jnp.float32)
            return 0
        lax.fori_loop(0, pad // B, mm, 0)


def _k2_body(dim, bases_ref, nch_ref, xb_ref, bcol_ref, xm2_ref, wgt_ref,
             bg_ref, lnw_ref, lnb_ref, pa_ref, out_ref, cat_ref, wcat_ref):
    k = pl.program_id(0)
    base = bases_ref[k]
    nch = nch_ref[k]
    bcol = bcol_ref[0]
    cols = lax.broadcasted_iota(jnp.int32, (B, C), 1)

    @pl.when(k == 0)
    def _():
        wcat_ref[:dim, :] = wgt_ref[...]

    cat_ref[:, :dim] = xb_ref[...]
    cat_ref[:, dim:] = jnp.where(bcol == cols + base, 1.0,
                                 0.0).astype(jnp.bfloat16)
    wcat_ref[dim:, :] = -xm2_ref[pl.ds(base, C), :].astype(jnp.bfloat16)

    out_ref[...] = jnp.dot(cat_ref[...], wcat_ref[...],
                           preferred_element_type=jnp.float32) + bg_ref[...]

    def chunk(j, _):
        cb = base + j * C
        oh = jnp.where(bcol == cols + cb, 1.0, 0.0).astype(jnp.bfloat16)
        out_ref[...] -= jnp.dot(
            oh, xm2_ref[pl.ds(cb, C), :].astype(jnp.bfloat16),
            preferred_element_type=jnp.float32)
        return 0

    lax.fori_loop(1, nch, chunk, 0)

    h = out_ref[...]
    mu = jnp.mean(h, axis=1, keepdims=True)
    d = h - mu
    var = jnp.mean(d * d, axis=1, keepdims=True)
    o = d * lax.rsqrt(var + 1e-5) * lnw_ref[...] + lnb_ref[...]
    out_ref[...] = jnp.where(o >= 0, o, pa_ref[0, 0] * o)


@functools.partial(jax.jit, static_argnames=("nseg", "interpret"))
def _run(x, batch, W_gamma, b_gamma, W_lambda, ln_w, ln_b, prelu_a,
         nseg=NSEG, interpret=False):
    n, dim = x.shape
    nblk = n // B
    pad = _pad_rows(nseg)

    batch32 = batch.astype(jnp.int32)
    starts = batch32[::B]
    ends = batch32[B - 1::B]
    bases = (starts // 8) * 8
    nch = (ends - bases) // C + 1
    brow = batch32.reshape(nblk, 1, B)
    bcol = batch32.reshape(nblk, B, 1)
    wlt = W_lambda.T
    wgt = W_gamma.T.astype(jnp.bfloat16)

    xm2, xb = pl.pallas_call(
        functools.partial(_k1_body, nblk, pad),
        grid_spec=pltpu.PrefetchScalarGridSpec(
            num_scalar_prefetch=2,
            grid=(nblk,),
            in_specs=[
                pl.BlockSpec((B, dim), lambda k, b, c: (k, 0)),
                pl.BlockSpec((1, 1, B), lambda k, b, c: (k, 0, 0)),
                pl.BlockSpec((dim, dim), lambda k, b, c: (0, 0)),
            ],
            out_specs=[
                pl.BlockSpec((pad, dim), lambda k, b, c: (0, 0)),
                pl.BlockSpec((B, dim), lambda k, b, c: (k, 0)),
            ],
            scratch_shapes=[pltpu.VMEM((pad, dim), jnp.float32)],
        ),
        out_shape=[
            jax.ShapeDtypeStruct((pad, dim), jnp.float32),
            jax.ShapeDtypeStruct((n, dim), jnp.bfloat16),
        ],
        interpret=interpret,
    )(bases, nch, x, brow, wlt)

    out = pl.pallas_call(
        functools.partial(_k2_body, dim),
        grid_spec=pltpu.PrefetchScalarGridSpec(
            num_scalar_prefetch=2,
            grid=(nblk,),
            in_specs=[
                pl.BlockSpec((B, dim), lambda k, b, c: (k, 0)),
                pl.BlockSpec((1, B, 1), lambda k, b, c: (k, 0, 0)),
                pl.BlockSpec((pad, dim), lambda k, b, c: (0, 0)),
                pl.BlockSpec((dim, dim), lambda k, b, c: (0, 0)),
                pl.BlockSpec((1, dim), lambda k, b, c: (0, 0)),
                pl.BlockSpec((1, dim), lambda k, b, c: (0, 0)),
                pl.BlockSpec((1, dim), lambda k, b, c: (0, 0)),
                pl.BlockSpec((1, 1), lambda k, b, c: (0, 0)),
            ],
            out_specs=pl.BlockSpec((B, dim), lambda k, b, c: (k, 0)),
            scratch_shapes=[
                pltpu.VMEM((B, dim + C), jnp.bfloat16),
                pltpu.VMEM((dim + C, dim), jnp.bfloat16),
            ],
        ),
        out_shape=jax.ShapeDtypeStruct((n, dim), jnp.float32),
        interpret=interpret,
    )(bases, nch, xb, bcol, xm2, wgt, b_gamma.reshape(1, dim),
      ln_w.reshape(1, dim), ln_b.reshape(1, dim), prelu_a.reshape(1, 1))
    return out


def kernel(x, batch, W_gamma, b_gamma, W_lambda, ln_w, ln_b, prelu_a):
    return _run(x, batch, W_gamma, b_gamma, W_lambda, ln_w, ln_b, prelu_a)


# speedup vs baseline: 1.7719x; 1.7719x over previous
import functools

import jax
import jax.numpy as jnp
from jax import lax
from jax.experimental import pallas as pl
from jax.experimental.pallas import tpu as pltpu

B = 2560
C = 128
NSEG = 10000


def _pad_rows(nseg):
    need = nseg + C + 8
    return ((need + B - 1) // B) * B


def _k1_body(nblk, pad, bases_ref, nch_ref, x_ref, brow_ref, wlt_ref,
             out_ref, xb_ref, acc_ref):
    k = pl.program_id(0)

    @pl.when(k == 0)
    def _():
        acc_ref[...] = jnp.zeros_like(acc_ref)

    base = bases_ref[k]
    nch = nch_ref[k]
    brow = brow_ref[0]
    x_blk = x_ref[...].astype(jnp.bfloat16)
    xb_ref[...] = x_blk

    def chunk(j, _):
        cb = base + j * C
        rows = lax.broadcasted_iota(jnp.int32, (C, B), 0) + cb
        ohT = jnp.where(rows == brow, 1.0, 0.0).astype(jnp.bfloat16)
        acc_ref[pl.ds(cb, C), :] += jnp.dot(
            ohT, x_blk, preferred_element_type=jnp.float32)
        return 0

    lax.fori_loop(0, nch, chunk, 0)

    @pl.when(k == nblk - 1)
    def _():
        def mm(i, _):
            out_ref[pl.ds(i * B, B), :] = jnp.dot(
                acc_ref[pl.ds(i * B, B), :], wlt_ref[...],
                preferred_element_type=jnp.float32)
            return 0
        lax.fori_loop(0, pad // B, mm, 0)


def _k2_body(dim, bases_ref, nch_ref, xb_ref, brow_ref, xm2_ref, wgt_ref,
             bg_ref, lnw_ref, lnb_ref, pa_ref, out_ref):
    k = pl.program_id(0)
    base = bases_ref[k]
    nch = nch_ref[k]
    brow = brow_ref[0]
    rows = lax.broadcasted_iota(jnp.int32, (C, B), 0)

    def gmm(cb):
        ohT = jnp.where(rows + cb == brow, 1.0, 0.0).astype(jnp.bfloat16)
        return lax.dot_general(
            ohT, xm2_ref[pl.ds(cb, C), :].astype(jnp.bfloat16),
            (((0,), (0,)), ((), ())), preferred_element_type=jnp.float32)

    out_ref[...] = (jnp.dot(xb_ref[...], wgt_ref[...],
                            preferred_element_type=jnp.float32)
                    + bg_ref[...] - gmm(base))

    def chunk(j, _):
        out_ref[...] -= gmm(base + j * C)
        return 0

    lax.fori_loop(1, nch, chunk, 0)

    h = out_ref[...]
    mu = jnp.mean(h, axis=1, keepdims=True)
    d = h - mu
    var = jnp.mean(d * d, axis=1, keepdims=True)
    o = d * lax.rsqrt(var + 1e-5) * lnw_ref[...] + lnb_ref[...]
    out_ref[...] = jnp.where(o >= 0, o, pa_ref[0, 0] * o)


@functools.partial(jax.jit, static_argnames=("nseg", "interpret"))
def _run(x, batch, W_gamma, b_gamma, W_lambda, ln_w, ln_b, prelu_a,
         nseg=NSEG, interpret=False):
    n, dim = x.shape
    nblk = n // B
    pad = _pad_rows(nseg)

    batch32 = batch.astype(jnp.int32)
    starts = batch32[::B]
    ends = batch32[B - 1::B]
    bases = (starts // 8) * 8
    nch = (ends - bases) // C + 1
    brow = batch32.reshape(nblk, 1, B)
    wlt = W_lambda.T
    wgt = W_gamma.T.astype(jnp.bfloat16)

    xm2, xb = pl.pallas_call(
        functools.partial(_k1_body, nblk, pad),
        grid_spec=pltpu.PrefetchScalarGridSpec(
            num_scalar_prefetch=2,
            grid=(nblk,),
            in_specs=[
                pl.BlockSpec((B, dim), lambda k, b, c: (k, 0)),
                pl.BlockSpec((1, 1, B), lambda k, b, c: (k, 0, 0)),
                pl.BlockSpec((dim, dim), lambda k, b, c: (0, 0)),
            ],
            out_specs=[
                pl.BlockSpec((pad, dim), lambda k, b, c: (0, 0)),
                pl.BlockSpec((B, dim), lambda k, b, c: (k, 0)),
            ],
            scratch_shapes=[pltpu.VMEM((pad, dim), jnp.float32)],
        ),
        out_shape=[
            jax.ShapeDtypeStruct((pad, dim), jnp.float32),
            jax.ShapeDtypeStruct((n, dim), jnp.bfloat16),
        ],
        interpret=interpret,
    )(bases, nch, x, brow, wlt)

    out = pl.pallas_call(
        functools.partial(_k2_body, dim),
        grid_spec=pltpu.PrefetchScalarGridSpec(
            num_scalar_prefetch=2,
            grid=(nblk,),
            in_specs=[
                pl.BlockSpec((B, dim), lambda k, b, c: (k, 0)),
                pl.BlockSpec((1, 1, B), lambda k, b, c: (k, 0, 0)),
                pl.BlockSpec((pad, dim), lambda k, b, c: (0, 0)),
                pl.BlockSpec((dim, dim), lambda k, b, c: (0, 0)),
                pl.BlockSpec((1, dim), lambda k, b, c: (0, 0)),
                pl.BlockSpec((1, dim), lambda k, b, c: (0, 0)),
                pl.BlockSpec((1, dim), lambda k, b, c: (0, 0)),
                pl.BlockSpec((1, 1), lambda k, b, c: (0, 0)),
            ],
            out_specs=pl.BlockSpec((B, dim), lambda k, b, c: (k, 0)),
        ),
        out_shape=jax.ShapeDtypeStruct((n, dim), jnp.float32),
        interpret=interpret,
    )(bases, nch, xb, brow, xm2, wgt, b_gamma.reshape(1, dim),
      ln_w.reshape(1, dim), ln_b.reshape(1, dim), prelu_a.reshape(1, 1))
    return out


def kernel(x, batch, W_gamma, b_gamma, W_lambda, ln_w, ln_b, prelu_a):
    return _run(x, batch, W_gamma, b_gamma, W_lambda, ln_w, ln_b, prelu_a)
